# Initial kernel scaffold; baseline (speedup 1.0000x reference)
import jax
import jax.numpy as jnp
from jax.experimental import pallas as pl


def kernel(x, edge_index, W_l, b_l, W_r):
    raise NotImplementedError("write your pallas kernel here")


# speedup vs baseline: 18.2901x; 18.2901x over previous
import functools

import jax
import jax.numpy as jnp
from jax import lax
from jax.experimental import pallas as pl
from jax.experimental.pallas import tpu as pltpu
from jax.experimental.pallas import tpu_sc as plsc

N_NODES = 100000
D = 16
E_EDGES = 3200000

NC, NS, L = 2, 16, 16
NW = NC * NS

CHUNK = 128
ROWS_PER_TILE = 49 * CHUNK
NPAD = NS * ROWS_PER_TILE
NPAD8 = NPAD // 8

SPT = 784
R = 16
OUTER = SPT // R
EPAD = NW * SPT * CHUNK
EC = EPAD // CHUNK

def _zero_acc(zbuf, acc, row0):
    for i in range(CHUNK):
        zbuf[i, :] = jnp.zeros((L,), jnp.float32)

    def zero_step(i, carry):
        pltpu.sync_copy(zbuf, acc.at[pl.ds(row0 + i * CHUNK, CHUNK)])
        return carry

    lax.fori_loop(0, ROWS_PER_TILE // CHUNK, zero_step, 0)


def _segsum_body(g_hbm, src_hbm, dst_hbm, out_hbm, srcb, dstb, rows, zbuf, acc, sem):
    c = lax.axis_index("c")
    s = lax.axis_index("s")
    row0 = s * ROWS_PER_TILE
    _zero_acc(zbuf, acc, row0)
    plsc.subcore_barrier()

    tile_blk0 = (c * NS + s) * SPT

    def outer_step(o, carry):
        blk = tile_blk0 + o * R
        pltpu.sync_copy(src_hbm.at[pl.ds(blk, R)], srcb)
        pltpu.sync_copy(dst_hbm.at[pl.ds(blk, R)], dstb)

        def inner_step(j, carry2):
            pltpu.async_copy(g_hbm.at[srcb.at[j]], rows, sem).wait()
            pltpu.sync_copy(rows, acc.at[dstb.at[j]], add=True)
            return carry2

        lax.fori_loop(0, R, inner_step, 0)
        return carry

    lax.fori_loop(0, OUTER, outer_step, 0)
    plsc.subcore_barrier()
    pltpu.sync_copy(
        acc.at[pl.ds(row0, ROWS_PER_TILE)],
        out_hbm.at[c, pl.ds(row0, ROWS_PER_TILE)],
    )


def _counts_body(dst_hbm, out_hbm, dstb, ones, zbuf, acc):
    c = lax.axis_index("c")
    s = lax.axis_index("s")
    row0 = s * ROWS_PER_TILE
    _zero_acc(zbuf, acc, row0)
    for i in range(CHUNK):
        ones[i, :] = jnp.ones((L,), jnp.float32)
    plsc.subcore_barrier()

    tile_blk0 = (c * NS + s) * SPT

    def outer_step(o, carry):
        blk = tile_blk0 + o * R
        pltpu.sync_copy(dst_hbm.at[pl.ds(blk, R)], dstb)

        def inner_step(j, carry2):
            pltpu.sync_copy(ones, acc.at[dstb.at[j]], add=True)
            return carry2

        lax.fori_loop(0, R, inner_step, 0)
        return carry

    lax.fori_loop(0, OUTER, outer_step, 0)
    plsc.subcore_barrier()
    pltpu.sync_copy(
        acc.at[pl.ds(row0, ROWS_PER_TILE)],
        out_hbm.at[c, pl.ds(row0, ROWS_PER_TILE)],
    )


@functools.cache
def _sc_kernels():
    mesh = plsc.VectorSubcoreMesh(
        core_axis_name="c", subcore_axis_name="s", num_cores=NC, num_subcores=NS
    )
    params = pltpu.CompilerParams(use_tc_tiling_on_sc=False)
    out_t = jax.ShapeDtypeStruct((NC, NPAD, D), jnp.float32)
    seg = pl.kernel(
        _segsum_body,
        out_type=out_t,
        mesh=mesh,
        compiler_params=params,
        scratch_types=[
            pltpu.VMEM((R, CHUNK), jnp.int32),
            pltpu.VMEM((R, CHUNK), jnp.int32),
            pltpu.VMEM((CHUNK, D), jnp.float32),
            pltpu.VMEM((CHUNK, D), jnp.float32),
            pltpu.VMEM_SHARED((NPAD, D), jnp.float32),
            pltpu.SemaphoreType.DMA,
        ],
    )
    cnt = pl.kernel(
        _counts_body,
        out_type=out_t,
        mesh=mesh,
        compiler_params=params,
        scratch_types=[
            pltpu.VMEM((R, CHUNK), jnp.int32),
            pltpu.VMEM((CHUNK, D), jnp.float32),
            pltpu.VMEM((CHUNK, D), jnp.float32),
            pltpu.VMEM_SHARED((NPAD, D), jnp.float32),
        ],
    )
    return seg, cnt


BN8 = 1792
_TC_GRID = NPAD8 // BN8

_node_spec = pl.BlockSpec((BN8, 128), lambda i: (i, 0))
_w_spec = pl.BlockSpec((128, 128), lambda i: (0, 0))
_b_spec = pl.BlockSpec((1, 128), lambda i: (0, 0))


def _prep_body(ca_ref, cb_ref, x_ref, wl_ref, inv_ref, g_ref):
    cnt = ca_ref[...] + cb_ref[...]
    inv_ref[...] = 1.0 / jnp.maximum(cnt, 1.0)
    g_ref[...] = jnp.dot(x_ref[...], wl_ref[...], preferred_element_type=jnp.float32)


_prep_tc = pl.pallas_call(
    _prep_body,
    grid=(_TC_GRID,),
    in_specs=[_node_spec, _node_spec, _node_spec, _w_spec],
    out_specs=[_node_spec, _node_spec],
    out_shape=[
        jax.ShapeDtypeStruct((NPAD8, 128), jnp.float32),
        jax.ShapeDtypeStruct((NPAD8, 128), jnp.float32),
    ],
)


def _round_body(aa_ref, ab_ref, inv_ref, h_ref, b_ref, wr_ref, wl_ref, h_out, g_out):
    m = (aa_ref[...] + ab_ref[...]) * inv_ref[...]
    hn = m + b_ref[...] + jnp.dot(
        h_ref[...], wr_ref[...], preferred_element_type=jnp.float32
    )
    h_out[...] = hn
    g_out[...] = jnp.dot(hn, wl_ref[...], preferred_element_type=jnp.float32)


_round_tc = pl.pallas_call(
    _round_body,
    grid=(_TC_GRID,),
    in_specs=[_node_spec, _node_spec, _node_spec, _node_spec, _b_spec, _w_spec, _w_spec],
    out_specs=[_node_spec, _node_spec],
    out_shape=[
        jax.ShapeDtypeStruct((NPAD8, 128), jnp.float32),
        jax.ShapeDtypeStruct((NPAD8, 128), jnp.float32),
    ],
)


def kernel(x, edge_index, W_l, b_l, W_r):
    src = edge_index[0]
    dst = edge_index[1]

    epad = EPAD - E_EDGES
    src2 = jnp.concatenate([src, jnp.zeros((epad,), jnp.int32)]).reshape(EC, CHUNK)
    dst2 = jnp.concatenate(
        [dst, jnp.full((epad,), N_NODES, jnp.int32)]
    ).reshape(EC, CHUNK)

    x_p = jnp.concatenate(
        [x, jnp.zeros((NPAD - N_NODES, D), jnp.float32)]
    )
    x8 = x_p.reshape(NPAD8, 128)

    eye8 = jnp.eye(8, dtype=jnp.float32)
    wl_big = jnp.kron(eye8, W_l.T)
    wr_big = jnp.kron(eye8, W_r.T)
    b_big = jnp.tile(b_l, 8).reshape(1, 128)

    segsum_sc, counts_sc = _sc_kernels()

    cnt = counts_sc(dst2)
    ca8 = cnt[0].reshape(NPAD8, 128)
    cb8 = cnt[1].reshape(NPAD8, 128)
    inv8, g8 = _prep_tc(ca8, cb8, x8, wl_big)

    hs = []
    h8 = x8
    for _ in range(3):
        acc = segsum_sc(g8.reshape(NPAD, D), src2, dst2)
        h8, g8 = _round_tc(
            acc[0].reshape(NPAD8, 128),
            acc[1].reshape(NPAD8, 128),
            inv8,
            h8,
            b_big,
            wr_big,
            wl_big,
        )
        hs.append(h8.reshape(NPAD, D)[:N_NODES])

    return jnp.concatenate([x] + hs, axis=-1)


# speedup vs baseline: 30.6878x; 1.6778x over previous
import functools

import jax
import jax.numpy as jnp
from jax import lax
from jax.experimental import pallas as pl
from jax.experimental.pallas import tpu as pltpu
from jax.experimental.pallas import tpu_sc as plsc

N_NODES = 100000
D = 16
E_EDGES = 3200000

NC, NS, L = 2, 16, 16
NW = NC * NS

CHUNK = 128
ROWS_PER_TILE = 49 * CHUNK
NPAD = NS * ROWS_PER_TILE
NPAD8 = NPAD // 8

SPT = 784
R = 16
OUTER = SPT // R
SBH = 4
NBODY = SPT // (2 * SBH)
EPAD = NW * SPT * CHUNK
EC = EPAD // CHUNK

def _zero_acc(zbuf, acc, row0):
    for i in range(CHUNK):
        zbuf[i, :] = jnp.zeros((L,), jnp.float32)

    def zero_step(i, carry):
        pltpu.sync_copy(
            zbuf.at[pl.ds(0, CHUNK)], acc.at[pl.ds(row0 + i * CHUNK, CHUNK)]
        )
        return carry

    lax.fori_loop(0, ROWS_PER_TILE // CHUNK, zero_step, 0)


def _segsum_body(
    g_hbm, src_hbm, dst_hbm, out_hbm,
    is0, id0, is1, id1, rows0, rows1, acc,
    semi0, semi1, semg0, semg1,
):
    c = lax.axis_index("c")
    s = lax.axis_index("s")
    row0 = s * ROWS_PER_TILE
    _zero_acc(rows0, acc, row0)
    plsc.subcore_barrier()

    tb = (c * NS + s) * SPT
    pltpu.async_copy(src_hbm.at[pl.ds(tb, SBH)], is0, semi0)
    pltpu.async_copy(dst_hbm.at[pl.ds(tb, SBH)], id0, semi0)
    pltpu.async_copy(src_hbm.at[pl.ds(tb + SBH, SBH)], is1, semi1)
    pltpu.async_copy(dst_hbm.at[pl.ds(tb + SBH, SBH)], id1, semi1)

    def body(p, carry):
        base = tb + p * (2 * SBH)
        pltpu.make_async_copy(src_hbm.at[pl.ds(base, SBH)], is0, semi0).wait()
        pltpu.make_async_copy(dst_hbm.at[pl.ds(base, SBH)], id0, semi0).wait()
        g0 = [
            pltpu.async_copy(
                g_hbm.at[is0.at[j]], rows0.at[pl.ds(j * CHUNK, CHUNK)], semg0
            )
            for j in range(SBH)
        ]
        pltpu.make_async_copy(src_hbm.at[pl.ds(base, SBH)], is1, semi1).wait()
        pltpu.make_async_copy(dst_hbm.at[pl.ds(base, SBH)], id1, semi1).wait()
        g1 = [
            pltpu.async_copy(
                g_hbm.at[is1.at[j]], rows1.at[pl.ds(j * CHUNK, CHUNK)], semg1
            )
            for j in range(SBH)
        ]

        for d in g0:
            d.wait()
        for j in range(SBH):
            pltpu.sync_copy(
                rows0.at[pl.ds(j * CHUNK, CHUNK)], acc.at[id0.at[j]], add=True
            )
        nbase = base + 2 * SBH

        @pl.when(p < NBODY - 1)
        def _():
            pltpu.async_copy(src_hbm.at[pl.ds(nbase, SBH)], is0, semi0)
            pltpu.async_copy(dst_hbm.at[pl.ds(nbase, SBH)], id0, semi0)

        for d in g1:
            d.wait()
        for j in range(SBH):
            pltpu.sync_copy(
                rows1.at[pl.ds(j * CHUNK, CHUNK)], acc.at[id1.at[j]], add=True
            )

        @pl.when(p < NBODY - 1)
        def _():
            pltpu.async_copy(src_hbm.at[pl.ds(nbase + SBH, SBH)], is1, semi1)
            pltpu.async_copy(dst_hbm.at[pl.ds(nbase + SBH, SBH)], id1, semi1)

        return carry

    lax.fori_loop(0, NBODY, body, 0)
    plsc.subcore_barrier()
    pltpu.sync_copy(
        acc.at[pl.ds(row0, ROWS_PER_TILE)],
        out_hbm.at[c, pl.ds(row0, ROWS_PER_TILE)],
    )


def _counts_body(dst_hbm, out_hbm, dstb, ones, zbuf, acc):
    c = lax.axis_index("c")
    s = lax.axis_index("s")
    row0 = s * ROWS_PER_TILE
    _zero_acc(zbuf, acc, row0)
    for i in range(CHUNK):
        ones[i, :] = jnp.ones((L,), jnp.float32)
    plsc.subcore_barrier()

    tile_blk0 = (c * NS + s) * SPT

    def outer_step(o, carry):
        blk = tile_blk0 + o * R
        pltpu.sync_copy(dst_hbm.at[pl.ds(blk, R)], dstb)

        def inner_step(j, carry2):
            pltpu.sync_copy(ones, acc.at[dstb.at[j]], add=True)
            return carry2

        lax.fori_loop(0, R, inner_step, 0)
        return carry

    lax.fori_loop(0, OUTER, outer_step, 0)
    plsc.subcore_barrier()
    pltpu.sync_copy(
        acc.at[pl.ds(row0, ROWS_PER_TILE)],
        out_hbm.at[c, pl.ds(row0, ROWS_PER_TILE)],
    )


@functools.cache
def _sc_kernels():
    mesh = plsc.VectorSubcoreMesh(
        core_axis_name="c", subcore_axis_name="s", num_cores=NC, num_subcores=NS
    )
    params = pltpu.CompilerParams(use_tc_tiling_on_sc=False)
    out_t = jax.ShapeDtypeStruct((NC, NPAD, D), jnp.float32)
    seg = pl.kernel(
        _segsum_body,
        out_type=out_t,
        mesh=mesh,
        compiler_params=params,
        scratch_types=[
            pltpu.VMEM((SBH, CHUNK), jnp.int32),
            pltpu.VMEM((SBH, CHUNK), jnp.int32),
            pltpu.VMEM((SBH, CHUNK), jnp.int32),
            pltpu.VMEM((SBH, CHUNK), jnp.int32),
            pltpu.VMEM((SBH * CHUNK, D), jnp.float32),
            pltpu.VMEM((SBH * CHUNK, D), jnp.float32),
            pltpu.VMEM_SHARED((NPAD, D), jnp.float32),
            pltpu.SemaphoreType.DMA,
            pltpu.SemaphoreType.DMA,
            pltpu.SemaphoreType.DMA,
            pltpu.SemaphoreType.DMA,
        ],
    )
    cnt = pl.kernel(
        _counts_body,
        out_type=out_t,
        mesh=mesh,
        compiler_params=params,
        scratch_types=[
            pltpu.VMEM((R, CHUNK), jnp.int32),
            pltpu.VMEM((CHUNK, D), jnp.float32),
            pltpu.VMEM((CHUNK, D), jnp.float32),
            pltpu.VMEM_SHARED((NPAD, D), jnp.float32),
        ],
    )
    return seg, cnt


BN8 = 1792
_TC_GRID = NPAD8 // BN8

_node_spec = pl.BlockSpec((BN8, 128), lambda i: (i, 0))
_w_spec = pl.BlockSpec((128, 128), lambda i: (0, 0))
_b_spec = pl.BlockSpec((1, 128), lambda i: (0, 0))


def _prep_body(ca_ref, cb_ref, x_ref, wl_ref, inv_ref, g_ref):
    cnt = ca_ref[...] + cb_ref[...]
    inv_ref[...] = 1.0 / jnp.maximum(cnt, 1.0)
    g_ref[...] = jnp.dot(x_ref[...], wl_ref[...], preferred_element_type=jnp.float32)


_prep_tc = pl.pallas_call(
    _prep_body,
    grid=(_TC_GRID,),
    in_specs=[_node_spec, _node_spec, _node_spec, _w_spec],
    out_specs=[_node_spec, _node_spec],
    out_shape=[
        jax.ShapeDtypeStruct((NPAD8, 128), jnp.float32),
        jax.ShapeDtypeStruct((NPAD8, 128), jnp.float32),
    ],
)


def _round_body(aa_ref, ab_ref, inv_ref, h_ref, b_ref, wr_ref, wl_ref, h_out, g_out):
    m = (aa_ref[...] + ab_ref[...]) * inv_ref[...]
    hn = m + b_ref[...] + jnp.dot(
        h_ref[...], wr_ref[...], preferred_element_type=jnp.float32
    )
    h_out[...] = hn
    g_out[...] = jnp.dot(hn, wl_ref[...], preferred_element_type=jnp.float32)


_round_tc = pl.pallas_call(
    _round_body,
    grid=(_TC_GRID,),
    in_specs=[_node_spec, _node_spec, _node_spec, _node_spec, _b_spec, _w_spec, _w_spec],
    out_specs=[_node_spec, _node_spec],
    out_shape=[
        jax.ShapeDtypeStruct((NPAD8, 128), jnp.float32),
        jax.ShapeDtypeStruct((NPAD8, 128), jnp.float32),
    ],
)


def kernel(x, edge_index, W_l, b_l, W_r):
    src = edge_index[0]
    dst = edge_index[1]

    epad = EPAD - E_EDGES
    src2 = jnp.concatenate([src, jnp.zeros((epad,), jnp.int32)]).reshape(EC, CHUNK)
    dst2 = jnp.concatenate(
        [dst, jnp.full((epad,), N_NODES, jnp.int32)]
    ).reshape(EC, CHUNK)

    x_p = jnp.concatenate(
        [x, jnp.zeros((NPAD - N_NODES, D), jnp.float32)]
    )
    x8 = x_p.reshape(NPAD8, 128)

    eye8 = jnp.eye(8, dtype=jnp.float32)
    wl_big = jnp.kron(eye8, W_l.T)
    wr_big = jnp.kron(eye8, W_r.T)
    b_big = jnp.tile(b_l, 8).reshape(1, 128)

    segsum_sc, counts_sc = _sc_kernels()

    cnt = counts_sc(dst2)
    ca8 = cnt[0].reshape(NPAD8, 128)
    cb8 = cnt[1].reshape(NPAD8, 128)
    inv8, g8 = _prep_tc(ca8, cb8, x8, wl_big)

    hs = []
    h8 = x8
    for _ in range(3):
        acc = segsum_sc(g8.reshape(NPAD, D), src2, dst2)
        h8, g8 = _round_tc(
            acc[0].reshape(NPAD8, 128),
            acc[1].reshape(NPAD8, 128),
            inv8,
            h8,
            b_big,
            wr_big,
            wl_big,
        )
        hs.append(h8.reshape(NPAD, D)[:N_NODES])

    return jnp.concatenate([x] + hs, axis=-1)


# speedup vs baseline: 31.4861x; 1.0260x over previous
import functools

import jax
import jax.numpy as jnp
from jax import lax
from jax.experimental import pallas as pl
from jax.experimental.pallas import tpu as pltpu
from jax.experimental.pallas import tpu_sc as plsc

N_NODES = 100000
D = 16
E_EDGES = 3200000

NC, NS, L = 2, 16, 16
NW = NC * NS

CHUNK = 128
ROWS_PER_TILE = 49 * CHUNK
NPAD = NS * ROWS_PER_TILE
NPAD8 = NPAD // 8

SPT = 784
R = 16
OUTER = SPT // R
SBH = 4
SBW = SBH * CHUNK
NBODY = SPT // (2 * SBH)
EPAD = NW * SPT * CHUNK
EC = EPAD // CHUNK

def _zero_acc(zbuf, acc, row0):
    for i in range(CHUNK):
        zbuf[i, :] = jnp.zeros((L,), jnp.float32)

    def zero_step(i, carry):
        pltpu.sync_copy(
            zbuf.at[pl.ds(0, CHUNK)], acc.at[pl.ds(row0 + i * CHUNK, CHUNK)]
        )
        return carry

    lax.fori_loop(0, ROWS_PER_TILE // CHUNK, zero_step, 0)


def _segsum_body(
    g_hbm, src_hbm, dst_hbm, out_hbm,
    is0, id0, is1, id1, rows0, rows1, acc,
    semi0, semi1, semg0, semg1,
):
    c = lax.axis_index("c")
    s = lax.axis_index("s")
    row0 = s * ROWS_PER_TILE
    _zero_acc(rows0, acc, row0)
    plsc.subcore_barrier()

    te = (c * NS + s) * (SPT * CHUNK)
    pltpu.async_copy(src_hbm.at[pl.ds(te, SBW)], is0, semi0)
    pltpu.async_copy(dst_hbm.at[pl.ds(te, SBW)], id0, semi0)
    pltpu.async_copy(src_hbm.at[pl.ds(te + SBW, SBW)], is1, semi1)
    pltpu.async_copy(dst_hbm.at[pl.ds(te + SBW, SBW)], id1, semi1)

    def body(p, carry):
        base = te + p * (2 * SBW)
        pltpu.make_async_copy(src_hbm.at[pl.ds(base, SBW)], is0, semi0).wait()
        pltpu.make_async_copy(dst_hbm.at[pl.ds(base, SBW)], id0, semi0).wait()
        g0 = pltpu.async_copy(g_hbm.at[is0], rows0, semg0)
        pltpu.make_async_copy(src_hbm.at[pl.ds(base, SBW)], is1, semi1).wait()
        pltpu.make_async_copy(dst_hbm.at[pl.ds(base, SBW)], id1, semi1).wait()
        g1 = pltpu.async_copy(g_hbm.at[is1], rows1, semg1)

        g0.wait()
        pltpu.sync_copy(rows0, acc.at[id0], add=True)
        nbase = base + 2 * SBW

        @pl.when(p < NBODY - 1)
        def _():
            pltpu.async_copy(src_hbm.at[pl.ds(nbase, SBW)], is0, semi0)
            pltpu.async_copy(dst_hbm.at[pl.ds(nbase, SBW)], id0, semi0)

        g1.wait()
        pltpu.sync_copy(rows1, acc.at[id1], add=True)

        @pl.when(p < NBODY - 1)
        def _():
            pltpu.async_copy(src_hbm.at[pl.ds(nbase + SBW, SBW)], is1, semi1)
            pltpu.async_copy(dst_hbm.at[pl.ds(nbase + SBW, SBW)], id1, semi1)

        return carry

    lax.fori_loop(0, NBODY, body, 0)
    plsc.subcore_barrier()
    pltpu.sync_copy(
        acc.at[pl.ds(row0, ROWS_PER_TILE)],
        out_hbm.at[c, pl.ds(row0, ROWS_PER_TILE)],
    )


def _counts_body(dst_hbm, out_hbm, dstb, ones, zbuf, acc):
    c = lax.axis_index("c")
    s = lax.axis_index("s")
    row0 = s * ROWS_PER_TILE
    _zero_acc(zbuf, acc, row0)
    for i in range(CHUNK):
        ones[i, :] = jnp.ones((L,), jnp.float32)
    plsc.subcore_barrier()

    tile_blk0 = (c * NS + s) * SPT

    def outer_step(o, carry):
        blk = tile_blk0 + o * R
        pltpu.sync_copy(dst_hbm.at[pl.ds(blk, R)], dstb)

        def inner_step(j, carry2):
            pltpu.sync_copy(ones, acc.at[dstb.at[j]], add=True)
            return carry2

        lax.fori_loop(0, R, inner_step, 0)
        return carry

    lax.fori_loop(0, OUTER, outer_step, 0)
    plsc.subcore_barrier()
    pltpu.sync_copy(
        acc.at[pl.ds(row0, ROWS_PER_TILE)],
        out_hbm.at[c, pl.ds(row0, ROWS_PER_TILE)],
    )


@functools.cache
def _sc_kernels():
    mesh = plsc.VectorSubcoreMesh(
        core_axis_name="c", subcore_axis_name="s", num_cores=NC, num_subcores=NS
    )
    params = pltpu.CompilerParams(use_tc_tiling_on_sc=False)
    out_t = jax.ShapeDtypeStruct((NC, NPAD, D), jnp.float32)
    seg = pl.kernel(
        _segsum_body,
        out_type=out_t,
        mesh=mesh,
        compiler_params=params,
        scratch_types=[
            pltpu.VMEM((SBW,), jnp.int32),
            pltpu.VMEM((SBW,), jnp.int32),
            pltpu.VMEM((SBW,), jnp.int32),
            pltpu.VMEM((SBW,), jnp.int32),
            pltpu.VMEM((SBW, D), jnp.float32),
            pltpu.VMEM((SBW, D), jnp.float32),
            pltpu.VMEM_SHARED((NPAD, D), jnp.float32),
            pltpu.SemaphoreType.DMA,
            pltpu.SemaphoreType.DMA,
            pltpu.SemaphoreType.DMA,
            pltpu.SemaphoreType.DMA,
        ],
    )
    cnt = pl.kernel(
        _counts_body,
        out_type=out_t,
        mesh=mesh,
        compiler_params=params,
        scratch_types=[
            pltpu.VMEM((R, CHUNK), jnp.int32),
            pltpu.VMEM((CHUNK, D), jnp.float32),
            pltpu.VMEM((CHUNK, D), jnp.float32),
            pltpu.VMEM_SHARED((NPAD, D), jnp.float32),
        ],
    )
    return seg, cnt


BN8 = 1792
_TC_GRID = NPAD8 // BN8

_node_spec = pl.BlockSpec((BN8, 128), lambda i: (i, 0))
_w_spec = pl.BlockSpec((128, 128), lambda i: (0, 0))
_b_spec = pl.BlockSpec((1, 128), lambda i: (0, 0))


def _prep_body(ca_ref, cb_ref, x_ref, wl_ref, inv_ref, g_ref):
    cnt = ca_ref[...] + cb_ref[...]
    inv_ref[...] = 1.0 / jnp.maximum(cnt, 1.0)
    g_ref[...] = jnp.dot(x_ref[...], wl_ref[...], preferred_element_type=jnp.float32)


_prep_tc = pl.pallas_call(
    _prep_body,
    grid=(_TC_GRID,),
    in_specs=[_node_spec, _node_spec, _node_spec, _w_spec],
    out_specs=[_node_spec, _node_spec],
    out_shape=[
        jax.ShapeDtypeStruct((NPAD8, 128), jnp.float32),
        jax.ShapeDtypeStruct((NPAD8, 128), jnp.float32),
    ],
)


def _round_body(aa_ref, ab_ref, inv_ref, h_ref, b_ref, wr_ref, wl_ref, h_out, g_out):
    m = (aa_ref[...] + ab_ref[...]) * inv_ref[...]
    hn = m + b_ref[...] + jnp.dot(
        h_ref[...], wr_ref[...], preferred_element_type=jnp.float32
    )
    h_out[...] = hn
    g_out[...] = jnp.dot(hn, wl_ref[...], preferred_element_type=jnp.float32)


_round_tc = pl.pallas_call(
    _round_body,
    grid=(_TC_GRID,),
    in_specs=[_node_spec, _node_spec, _node_spec, _node_spec, _b_spec, _w_spec, _w_spec],
    out_specs=[_node_spec, _node_spec],
    out_shape=[
        jax.ShapeDtypeStruct((NPAD8, 128), jnp.float32),
        jax.ShapeDtypeStruct((NPAD8, 128), jnp.float32),
    ],
)


def kernel(x, edge_index, W_l, b_l, W_r):
    src = edge_index[0]
    dst = edge_index[1]

    epad = EPAD - E_EDGES
    src1 = jnp.concatenate([src, jnp.zeros((epad,), jnp.int32)])
    dst1 = jnp.concatenate([dst, jnp.full((epad,), N_NODES, jnp.int32)])
    dst2 = dst1.reshape(EC, CHUNK)

    x_p = jnp.concatenate(
        [x, jnp.zeros((NPAD - N_NODES, D), jnp.float32)]
    )
    x8 = x_p.reshape(NPAD8, 128)

    eye8 = jnp.eye(8, dtype=jnp.float32)
    wl_big = jnp.kron(eye8, W_l.T)
    wr_big = jnp.kron(eye8, W_r.T)
    b_big = jnp.tile(b_l, 8).reshape(1, 128)

    segsum_sc, counts_sc = _sc_kernels()

    cnt = counts_sc(dst2)
    ca8 = cnt[0].reshape(NPAD8, 128)
    cb8 = cnt[1].reshape(NPAD8, 128)
    inv8, g8 = _prep_tc(ca8, cb8, x8, wl_big)

    hs = []
    h8 = x8
    for _ in range(3):
        acc = segsum_sc(g8.reshape(NPAD, D), src1, dst1)
        h8, g8 = _round_tc(
            acc[0].reshape(NPAD8, 128),
            acc[1].reshape(NPAD8, 128),
            inv8,
            h8,
            b_big,
            wr_big,
            wl_big,
        )
        hs.append(h8.reshape(NPAD, D)[:N_NODES])

    return jnp.concatenate([x] + hs, axis=-1)


# speedup vs baseline: 48.6225x; 1.5443x over previous
import functools

import jax
import jax.numpy as jnp
from jax import lax
from jax.experimental import pallas as pl
from jax.experimental.pallas import tpu as pltpu
from jax.experimental.pallas import tpu_sc as plsc

N_NODES = 100000
D = 16
E_EDGES = 3200000

NC, NS, L = 2, 16, 16
NW = NC * NS

CHUNK = 128
ROWS_PER_TILE = 49 * CHUNK
NPAD = NS * ROWS_PER_TILE
NPAD8 = NPAD // 8

SPT = 784
R = 16
OUTER = SPT // R
SBH = 4
SBW = SBH * CHUNK
NBODY = SPT // (2 * SBH)
EPAD = NW * SPT * CHUNK
EC = EPAD // CHUNK

def _zero_acc(zbuf, acc, row0):
    for i in range(CHUNK):
        zbuf[i, :] = jnp.zeros((L,), jnp.float32)

    def zero_step(i, carry):
        pltpu.sync_copy(
            zbuf.at[pl.ds(0, CHUNK)], acc.at[pl.ds(row0 + i * CHUNK, CHUNK)]
        )
        return carry

    lax.fori_loop(0, ROWS_PER_TILE // CHUNK, zero_step, 0)


def _segsum_body(
    g_hbm, src_hbm, dst_hbm, out_hbm,
    is0, id0, is1, id1, rows0, rows1, acc,
    semi0, semi1, semg0, semg1,
):
    c = lax.axis_index("c")
    s = lax.axis_index("s")
    row0 = s * ROWS_PER_TILE
    _zero_acc(rows0, acc, row0)
    plsc.subcore_barrier()

    te = (c * NS + s) * (SPT * CHUNK)
    pltpu.async_copy(src_hbm.at[pl.ds(te, SBW)], is0, semi0)
    pltpu.async_copy(dst_hbm.at[pl.ds(te, SBW)], id0, semi0)
    pltpu.async_copy(src_hbm.at[pl.ds(te + SBW, SBW)], is1, semi1)
    pltpu.async_copy(dst_hbm.at[pl.ds(te + SBW, SBW)], id1, semi1)

    def body(p, carry):
        base = te + p * (2 * SBW)
        pltpu.make_async_copy(src_hbm.at[pl.ds(base, SBW)], is0, semi0).wait()
        pltpu.make_async_copy(dst_hbm.at[pl.ds(base, SBW)], id0, semi0).wait()
        g0 = pltpu.async_copy(g_hbm.at[is0], rows0, semg0)
        pltpu.make_async_copy(src_hbm.at[pl.ds(base, SBW)], is1, semi1).wait()
        pltpu.make_async_copy(dst_hbm.at[pl.ds(base, SBW)], id1, semi1).wait()
        g1 = pltpu.async_copy(g_hbm.at[is1], rows1, semg1)

        g0.wait()
        pltpu.sync_copy(rows0, acc.at[id0], add=True)
        nbase = base + 2 * SBW

        @pl.when(p < NBODY - 1)
        def _():
            pltpu.async_copy(src_hbm.at[pl.ds(nbase, SBW)], is0, semi0)
            pltpu.async_copy(dst_hbm.at[pl.ds(nbase, SBW)], id0, semi0)

        g1.wait()
        pltpu.sync_copy(rows1, acc.at[id1], add=True)

        @pl.when(p < NBODY - 1)
        def _():
            pltpu.async_copy(src_hbm.at[pl.ds(nbase + SBW, SBW)], is1, semi1)
            pltpu.async_copy(dst_hbm.at[pl.ds(nbase + SBW, SBW)], id1, semi1)

        return carry

    lax.fori_loop(0, NBODY, body, 0)
    plsc.subcore_barrier()
    pltpu.sync_copy(
        acc.at[pl.ds(row0, ROWS_PER_TILE)],
        out_hbm.at[c, pl.ds(row0, ROWS_PER_TILE)],
    )


def _counts_body(dst_hbm, out_hbm, dstb, ones, zbuf, acc):
    c = lax.axis_index("c")
    s = lax.axis_index("s")
    row0 = s * ROWS_PER_TILE
    _zero_acc(zbuf, acc, row0)
    for i in range(CHUNK):
        ones[i, :] = jnp.ones((L,), jnp.float32)
    plsc.subcore_barrier()

    tile_blk0 = (c * NS + s) * SPT

    def outer_step(o, carry):
        blk = tile_blk0 + o * R
        pltpu.sync_copy(dst_hbm.at[pl.ds(blk, R)], dstb)

        def inner_step(j, carry2):
            pltpu.sync_copy(ones, acc.at[dstb.at[j]], add=True)
            return carry2

        lax.fori_loop(0, R, inner_step, 0)
        return carry

    lax.fori_loop(0, OUTER, outer_step, 0)
    plsc.subcore_barrier()
    pltpu.sync_copy(
        acc.at[pl.ds(row0, ROWS_PER_TILE)],
        out_hbm.at[c, pl.ds(row0, ROWS_PER_TILE)],
    )


@functools.cache
def _sc_kernels():
    mesh = plsc.VectorSubcoreMesh(
        core_axis_name="c", subcore_axis_name="s", num_cores=NC, num_subcores=NS
    )
    params = pltpu.CompilerParams(use_tc_tiling_on_sc=False)
    out_t = jax.ShapeDtypeStruct((NC, NPAD, D), jnp.float32)
    seg = pl.kernel(
        _segsum_body,
        out_type=out_t,
        mesh=mesh,
        compiler_params=params,
        scratch_types=[
            pltpu.VMEM((SBW,), jnp.int32),
            pltpu.VMEM((SBW,), jnp.int32),
            pltpu.VMEM((SBW,), jnp.int32),
            pltpu.VMEM((SBW,), jnp.int32),
            pltpu.VMEM((SBW, D), jnp.float32),
            pltpu.VMEM((SBW, D), jnp.float32),
            pltpu.VMEM_SHARED((NPAD, D), jnp.float32),
            pltpu.SemaphoreType.DMA,
            pltpu.SemaphoreType.DMA,
            pltpu.SemaphoreType.DMA,
            pltpu.SemaphoreType.DMA,
        ],
    )
    cnt = pl.kernel(
        _counts_body,
        out_type=out_t,
        mesh=mesh,
        compiler_params=params,
        scratch_types=[
            pltpu.VMEM((R, CHUNK), jnp.int32),
            pltpu.VMEM((CHUNK, D), jnp.float32),
            pltpu.VMEM((CHUNK, D), jnp.float32),
            pltpu.VMEM_SHARED((NPAD, D), jnp.float32),
        ],
    )
    return seg, cnt


BN8 = 1792
_TC_GRID = NPAD8 // BN8

_node_spec = pl.BlockSpec((BN8, 128), lambda i: (i, 0))
_plane0_spec = pl.BlockSpec((BN8, 128), lambda i: (i, 0))
_plane1_spec = pl.BlockSpec((BN8, 128), lambda i: (i + _TC_GRID, 0))
_w_spec = pl.BlockSpec((128, 128), lambda i: (0, 0))
_b_spec = pl.BlockSpec((1, 128), lambda i: (0, 0))


def _prep_body(ca_ref, cb_ref, x_ref, wl_ref, inv_ref, g_ref):
    cnt = ca_ref[...] + cb_ref[...]
    inv_ref[...] = 1.0 / jnp.maximum(cnt, 1.0)
    g_ref[...] = jnp.dot(x_ref[...], wl_ref[...], preferred_element_type=jnp.float32)


_prep_tc = pl.pallas_call(
    _prep_body,
    grid=(_TC_GRID,),
    in_specs=[_plane0_spec, _plane1_spec, _node_spec, _w_spec],
    out_specs=[_node_spec, _node_spec],
    out_shape=[
        jax.ShapeDtypeStruct((NPAD8, 128), jnp.float32),
        jax.ShapeDtypeStruct((NPAD8, 128), jnp.float32),
    ],
)


def _round_body(aa_ref, ab_ref, inv_ref, h_ref, b_ref, wr_ref, wl_ref, h_out, g_out):
    m = (aa_ref[...] + ab_ref[...]) * inv_ref[...]
    hn = m + b_ref[...] + jnp.dot(
        h_ref[...], wr_ref[...], preferred_element_type=jnp.float32
    )
    h_out[...] = hn
    g_out[...] = jnp.dot(hn, wl_ref[...], preferred_element_type=jnp.float32)


_round_tc = pl.pallas_call(
    _round_body,
    grid=(_TC_GRID,),
    in_specs=[_plane0_spec, _plane1_spec, _node_spec, _node_spec, _b_spec, _w_spec, _w_spec],
    out_specs=[_node_spec, _node_spec],
    out_shape=[
        jax.ShapeDtypeStruct((NPAD8, 128), jnp.float32),
        jax.ShapeDtypeStruct((NPAD8, 128), jnp.float32),
    ],
)


def kernel(x, edge_index, W_l, b_l, W_r):
    src = edge_index[0]
    dst = edge_index[1]

    epad = EPAD - E_EDGES
    src1 = jnp.concatenate([src, jnp.zeros((epad,), jnp.int32)])
    dst1 = jnp.concatenate([dst, jnp.full((epad,), N_NODES, jnp.int32)])
    dst2 = dst1.reshape(EC, CHUNK)

    x_p = jnp.concatenate(
        [x, jnp.zeros((NPAD - N_NODES, D), jnp.float32)]
    )
    x8 = x_p.reshape(NPAD8, 128)

    eye8 = jnp.eye(8, dtype=jnp.float32)
    wl_big = jnp.kron(eye8, W_l.T)
    wr_big = jnp.kron(eye8, W_r.T)
    b_big = jnp.tile(b_l, 8).reshape(1, 128)

    segsum_sc, counts_sc = _sc_kernels()

    cnt2 = counts_sc(dst2).reshape(2 * NPAD8, 128)
    inv8, g8 = _prep_tc(cnt2, cnt2, x8, wl_big)

    hs = []
    h8 = x8
    for _ in range(3):
        acc2 = segsum_sc(g8.reshape(NPAD, D), src1, dst1).reshape(2 * NPAD8, 128)
        h8, g8 = _round_tc(
            acc2,
            acc2,
            inv8,
            h8,
            b_big,
            wr_big,
            wl_big,
        )
        hs.append(h8.reshape(NPAD, D)[:N_NODES])

    return jnp.concatenate([x] + hs, axis=-1)


# speedup vs baseline: 54.6623x; 1.1242x over previous
import functools

import jax
import jax.numpy as jnp
from jax import lax
from jax.experimental import pallas as pl
from jax.experimental.pallas import tpu as pltpu
from jax.experimental.pallas import tpu_sc as plsc

N_NODES = 100000
D = 16
E_EDGES = 3200000

NC, NS, L = 2, 16, 16
NW = NC * NS

CHUNK = 128
ROWS_PER_TILE = 49 * CHUNK
NPAD = NS * ROWS_PER_TILE
NPAD8 = NPAD // 8

SPT = 784
R = 16
OUTER = SPT // R
SBW = 256
BANKS = 4
NBODY = SPT * CHUNK // (BANKS * SBW)
EPAD = NW * SPT * CHUNK
EC = EPAD // CHUNK

def _zero_acc(zbuf, acc, row0):
    for i in range(CHUNK):
        zbuf[i, :] = jnp.zeros((L,), jnp.float32)

    def zero_step(i, carry):
        pltpu.sync_copy(
            zbuf.at[pl.ds(0, CHUNK)], acc.at[pl.ds(row0 + i * CHUNK, CHUNK)]
        )
        return carry

    lax.fori_loop(0, ROWS_PER_TILE // CHUNK, zero_step, 0)


def _segsum_body(
    g_hbm, src_hbm, dst_hbm, out_hbm,
    iss, ids, idscs, rowss, acc, semis, semgs, semss,
):
    c = lax.axis_index("c")
    s = lax.axis_index("s")
    row0 = s * ROWS_PER_TILE
    _zero_acc(rowss[0], acc, row0)
    plsc.subcore_barrier()

    te = (c * NS + s) * (SPT * CHUNK)
    for k in range(BANKS):
        pltpu.async_copy(src_hbm.at[pl.ds(te + k * SBW, SBW)], iss[k], semis[k])
        pltpu.async_copy(dst_hbm.at[pl.ds(te + k * SBW, SBW)], ids[k], semis[k])

    def body(p, carry):
        base = te + p * (BANKS * SBW)
        gs = []
        for k in range(BANKS):
            eb = base + k * SBW
            pltpu.make_async_copy(src_hbm.at[pl.ds(eb, SBW)], iss[k], semis[k]).wait()
            pltpu.make_async_copy(dst_hbm.at[pl.ds(eb, SBW)], ids[k], semis[k]).wait()

            @pl.when(p > 0)
            def _(k=k):
                pltpu.make_async_copy(
                    g_hbm.at[pl.ds(0, SBW)], rowss[k], semss[k]
                ).wait()

            gs.append(pltpu.async_copy(g_hbm.at[iss[k]], rowss[k], semgs[k]))

        for k in range(BANKS):
            gs[k].wait()
            for i in range(SBW // L):
                idscs[k][pl.ds(i * L, L)] = ids[k][pl.ds(i * L, L)]
            pltpu.async_copy(rowss[k], acc.at[idscs[k]], semss[k], add=True)

            @pl.when(p < NBODY - 1)
            def _(k=k):
                nb = base + BANKS * SBW + k * SBW
                pltpu.async_copy(src_hbm.at[pl.ds(nb, SBW)], iss[k], semis[k])
                pltpu.async_copy(dst_hbm.at[pl.ds(nb, SBW)], ids[k], semis[k])

        return carry

    lax.fori_loop(0, NBODY, body, 0)
    for k in range(BANKS):
        pltpu.make_async_copy(g_hbm.at[pl.ds(0, SBW)], rowss[k], semss[k]).wait()
    plsc.subcore_barrier()
    pltpu.sync_copy(
        acc.at[pl.ds(row0, ROWS_PER_TILE)],
        out_hbm.at[c, pl.ds(row0, ROWS_PER_TILE)],
    )


def _counts_body(dst_hbm, out_hbm, dstb, ones, zbuf, acc):
    c = lax.axis_index("c")
    s = lax.axis_index("s")
    row0 = s * ROWS_PER_TILE
    _zero_acc(zbuf, acc, row0)
    for i in range(CHUNK):
        ones[i, :] = jnp.ones((L,), jnp.float32)
    plsc.subcore_barrier()

    tile_blk0 = (c * NS + s) * SPT

    def outer_step(o, carry):
        blk = tile_blk0 + o * R
        pltpu.sync_copy(dst_hbm.at[pl.ds(blk, R)], dstb)

        def inner_step(j, carry2):
            pltpu.sync_copy(ones, acc.at[dstb.at[j]], add=True)
            return carry2

        lax.fori_loop(0, R, inner_step, 0)
        return carry

    lax.fori_loop(0, OUTER, outer_step, 0)
    plsc.subcore_barrier()
    pltpu.sync_copy(
        acc.at[pl.ds(row0, ROWS_PER_TILE)],
        out_hbm.at[c, pl.ds(row0, ROWS_PER_TILE)],
    )


@functools.cache
def _sc_kernels():
    mesh = plsc.VectorSubcoreMesh(
        core_axis_name="c", subcore_axis_name="s", num_cores=NC, num_subcores=NS
    )
    params = pltpu.CompilerParams(use_tc_tiling_on_sc=False)
    out_t = jax.ShapeDtypeStruct((NC, NPAD, D), jnp.float32)
    seg = pl.kernel(
        _segsum_body,
        out_type=out_t,
        mesh=mesh,
        compiler_params=params,
        scratch_types=[
            [pltpu.VMEM((SBW,), jnp.int32) for _ in range(BANKS)],
            [pltpu.VMEM((SBW,), jnp.int32) for _ in range(BANKS)],
            [pltpu.VMEM((SBW,), jnp.int32) for _ in range(BANKS)],
            [pltpu.VMEM((SBW, D), jnp.float32) for _ in range(BANKS)],
            pltpu.VMEM_SHARED((NPAD, D), jnp.float32),
            [pltpu.SemaphoreType.DMA for _ in range(BANKS)],
            [pltpu.SemaphoreType.DMA for _ in range(BANKS)],
            [pltpu.SemaphoreType.DMA for _ in range(BANKS)],
        ],
    )
    cnt = pl.kernel(
        _counts_body,
        out_type=out_t,
        mesh=mesh,
        compiler_params=params,
        scratch_types=[
            pltpu.VMEM((R, CHUNK), jnp.int32),
            pltpu.VMEM((CHUNK, D), jnp.float32),
            pltpu.VMEM((CHUNK, D), jnp.float32),
            pltpu.VMEM_SHARED((NPAD, D), jnp.float32),
        ],
    )
    return seg, cnt


BN8 = 1792
_TC_GRID = NPAD8 // BN8

_node_spec = pl.BlockSpec((BN8, 128), lambda i: (i, 0))
_plane0_spec = pl.BlockSpec((BN8, 128), lambda i: (i, 0))
_plane1_spec = pl.BlockSpec((BN8, 128), lambda i: (i + _TC_GRID, 0))
_w_spec = pl.BlockSpec((128, 128), lambda i: (0, 0))
_b_spec = pl.BlockSpec((1, 128), lambda i: (0, 0))


def _prep_body(ca_ref, cb_ref, x_ref, wl_ref, inv_ref, g_ref):
    cnt = ca_ref[...] + cb_ref[...]
    inv_ref[...] = 1.0 / jnp.maximum(cnt, 1.0)
    g_ref[...] = jnp.dot(x_ref[...], wl_ref[...], preferred_element_type=jnp.float32)


_prep_tc = pl.pallas_call(
    _prep_body,
    grid=(_TC_GRID,),
    in_specs=[_plane0_spec, _plane1_spec, _node_spec, _w_spec],
    out_specs=[_node_spec, _node_spec],
    out_shape=[
        jax.ShapeDtypeStruct((NPAD8, 128), jnp.float32),
        jax.ShapeDtypeStruct((NPAD8, 128), jnp.float32),
    ],
)


def _round_body(aa_ref, ab_ref, inv_ref, h_ref, b_ref, wr_ref, wl_ref, h_out, g_out):
    m = (aa_ref[...] + ab_ref[...]) * inv_ref[...]
    hn = m + b_ref[...] + jnp.dot(
        h_ref[...], wr_ref[...], preferred_element_type=jnp.float32
    )
    h_out[...] = hn
    g_out[...] = jnp.dot(hn, wl_ref[...], preferred_element_type=jnp.float32)


_round_tc = pl.pallas_call(
    _round_body,
    grid=(_TC_GRID,),
    in_specs=[_plane0_spec, _plane1_spec, _node_spec, _node_spec, _b_spec, _w_spec, _w_spec],
    out_specs=[_node_spec, _node_spec],
    out_shape=[
        jax.ShapeDtypeStruct((NPAD8, 128), jnp.float32),
        jax.ShapeDtypeStruct((NPAD8, 128), jnp.float32),
    ],
)


def kernel(x, edge_index, W_l, b_l, W_r):
    src = edge_index[0]
    dst = edge_index[1]

    epad = EPAD - E_EDGES
    src1 = jnp.concatenate([src, jnp.zeros((epad,), jnp.int32)])
    dst1 = jnp.concatenate([dst, jnp.full((epad,), N_NODES, jnp.int32)])
    dst2 = dst1.reshape(EC, CHUNK)

    x_p = jnp.concatenate(
        [x, jnp.zeros((NPAD - N_NODES, D), jnp.float32)]
    )
    x8 = x_p.reshape(NPAD8, 128)

    eye8 = jnp.eye(8, dtype=jnp.float32)
    wl_big = jnp.kron(eye8, W_l.T)
    wr_big = jnp.kron(eye8, W_r.T)
    b_big = jnp.tile(b_l, 8).reshape(1, 128)

    segsum_sc, counts_sc = _sc_kernels()

    cnt2 = counts_sc(dst2).reshape(2 * NPAD8, 128)
    inv8, g8 = _prep_tc(cnt2, cnt2, x8, wl_big)

    hs = []
    h8 = x8
    for _ in range(3):
        acc2 = segsum_sc(g8.reshape(NPAD, D), src1, dst1).reshape(2 * NPAD8, 128)
        h8, g8 = _round_tc(
            acc2,
            acc2,
            inv8,
            h8,
            b_big,
            wr_big,
            wl_big,
        )
        hs.append(h8.reshape(NPAD, D)[:N_NODES])

    return jnp.concatenate([x] + hs, axis=-1)


# speedup vs baseline: 62.4705x; 1.1428x over previous
import functools

import jax
import jax.numpy as jnp
from jax import lax
from jax.experimental import pallas as pl
from jax.experimental.pallas import tpu as pltpu
from jax.experimental.pallas import tpu_sc as plsc

N_NODES = 100000
D = 16
E_EDGES = 3200000

NC, NS, L = 2, 16, 16
NW = NC * NS

CHUNK = 128
ROWS_PER_TILE = 49 * CHUNK
NPAD = NS * ROWS_PER_TILE
NPAD8 = NPAD // 8

SPT = 784
R = 16
OUTER = SPT // R
SBW = 256
BANKS = 4
EPT = SPT * CHUNK
NBODY = EPT // (BANKS * SBW)
LAST_E = E_EDGES - (NW - 1) * EPT
LAST_NB_SEG = LAST_E // (BANKS * SBW)
CNT_SBW = 512
CNT_NB = EPT // CNT_SBW
LAST_NB_CNT = LAST_E // CNT_SBW
EPAD = NW * SPT * CHUNK
EC = EPAD // CHUNK

def _zero_acc(zbuf, acc, row0):
    for i in range(CHUNK):
        zbuf[i, :] = jnp.zeros((L,), jnp.float32)

    def zero_step(i, carry):
        pltpu.sync_copy(
            zbuf.at[pl.ds(0, CHUNK)], acc.at[pl.ds(row0 + i * CHUNK, CHUNK)]
        )
        return carry

    lax.fori_loop(0, ROWS_PER_TILE // CHUNK, zero_step, 0)


def _segsum_body(
    g_hbm, ei_hbm, out_hbm,
    iss, ids, idscs, rowss, acc, semis, semgs, semss,
):
    c = lax.axis_index("c")
    s = lax.axis_index("s")
    row0 = s * ROWS_PER_TILE
    _zero_acc(rowss[0], acc, row0)
    plsc.subcore_barrier()

    w = c * NS + s
    te = w * (SPT * CHUNK)
    nbody = jnp.where(w == NW - 1, LAST_NB_SEG, NBODY)
    for k in range(BANKS):
        pltpu.async_copy(ei_hbm.at[0, pl.ds(te + k * SBW, SBW)], iss[k], semis[k])
        pltpu.async_copy(ei_hbm.at[1, pl.ds(te + k * SBW, SBW)], ids[k], semis[k])

    def body(p, carry):
        base = te + p * (BANKS * SBW)
        gs = []
        for k in range(BANKS):
            eb = base + k * SBW
            pltpu.make_async_copy(ei_hbm.at[0, pl.ds(eb, SBW)], iss[k], semis[k]).wait()
            pltpu.make_async_copy(ei_hbm.at[1, pl.ds(eb, SBW)], ids[k], semis[k]).wait()

            @pl.when(p > 0)
            def _(k=k):
                pltpu.make_async_copy(
                    g_hbm.at[pl.ds(0, SBW)], rowss[k], semss[k]
                ).wait()

            gs.append(pltpu.async_copy(g_hbm.at[iss[k]], rowss[k], semgs[k]))

        for k in range(BANKS):
            gs[k].wait()
            for i in range(SBW // L):
                idscs[k][pl.ds(i * L, L)] = ids[k][pl.ds(i * L, L)]
            pltpu.async_copy(rowss[k], acc.at[idscs[k]], semss[k], add=True)

            @pl.when(p < nbody - 1)
            def _(k=k):
                nb = base + BANKS * SBW + k * SBW
                pltpu.async_copy(ei_hbm.at[0, pl.ds(nb, SBW)], iss[k], semis[k])
                pltpu.async_copy(ei_hbm.at[1, pl.ds(nb, SBW)], ids[k], semis[k])

        return carry

    lax.fori_loop(0, nbody, body, 0)
    for k in range(BANKS):
        pltpu.make_async_copy(g_hbm.at[pl.ds(0, SBW)], rowss[k], semss[k]).wait()
    plsc.subcore_barrier()
    pltpu.sync_copy(
        acc.at[pl.ds(row0, ROWS_PER_TILE)],
        out_hbm.at[c, pl.ds(row0, ROWS_PER_TILE)],
    )


def _counts_body(ei_hbm, out_hbm, dstb, ones, acc):
    c = lax.axis_index("c")
    s = lax.axis_index("s")
    row0 = s * ROWS_PER_TILE
    _zero_acc(ones, acc, row0)
    for i in range(CNT_SBW):
        ones[i, :] = jnp.ones((L,), jnp.float32)
    plsc.subcore_barrier()

    w = c * NS + s
    te = w * (SPT * CHUNK)
    nbody = jnp.where(w == NW - 1, LAST_NB_CNT, CNT_NB)

    def outer_step(o, carry):
        pltpu.sync_copy(ei_hbm.at[1, pl.ds(te + o * CNT_SBW, CNT_SBW)], dstb)
        pltpu.sync_copy(ones, acc.at[dstb], add=True)
        return carry

    lax.fori_loop(0, nbody, outer_step, 0)
    plsc.subcore_barrier()
    pltpu.sync_copy(
        acc.at[pl.ds(row0, ROWS_PER_TILE)],
        out_hbm.at[c, pl.ds(row0, ROWS_PER_TILE)],
    )


@functools.cache
def _sc_kernels():
    mesh = plsc.VectorSubcoreMesh(
        core_axis_name="c", subcore_axis_name="s", num_cores=NC, num_subcores=NS
    )
    params = pltpu.CompilerParams(use_tc_tiling_on_sc=False)
    out_t = jax.ShapeDtypeStruct((NC, NPAD, D), jnp.float32)
    seg = pl.kernel(
        _segsum_body,
        out_type=out_t,
        mesh=mesh,
        compiler_params=params,
        scratch_types=[
            [pltpu.VMEM((SBW,), jnp.int32) for _ in range(BANKS)],
            [pltpu.VMEM((SBW,), jnp.int32) for _ in range(BANKS)],
            [pltpu.VMEM((SBW,), jnp.int32) for _ in range(BANKS)],
            [pltpu.VMEM((SBW, D), jnp.float32) for _ in range(BANKS)],
            pltpu.VMEM_SHARED((NPAD, D), jnp.float32),
            [pltpu.SemaphoreType.DMA for _ in range(BANKS)],
            [pltpu.SemaphoreType.DMA for _ in range(BANKS)],
            [pltpu.SemaphoreType.DMA for _ in range(BANKS)],
        ],
    )
    cnt = pl.kernel(
        _counts_body,
        out_type=out_t,
        mesh=mesh,
        compiler_params=params,
        scratch_types=[
            pltpu.VMEM((CNT_SBW,), jnp.int32),
            pltpu.VMEM((CNT_SBW, D), jnp.float32),
            pltpu.VMEM_SHARED((NPAD, D), jnp.float32),
        ],
    )
    return seg, cnt


BN8 = 1792
_TC_GRID = NPAD8 // BN8

_node_spec = pl.BlockSpec((BN8, 128), lambda i: (i, 0))
_plane0_spec = pl.BlockSpec((BN8, 128), lambda i: (i, 0))
_plane1_spec = pl.BlockSpec((BN8, 128), lambda i: (i + _TC_GRID, 0))
_w_spec = pl.BlockSpec((128, 128), lambda i: (0, 0))
_b_spec = pl.BlockSpec((1, 128), lambda i: (0, 0))


def _prep_body(ca_ref, cb_ref, x_ref, wl_ref, inv_ref, g_ref):
    cnt = ca_ref[...] + cb_ref[...]
    inv_ref[...] = 1.0 / jnp.maximum(cnt, 1.0)
    g_ref[...] = jnp.dot(x_ref[...], wl_ref[...], preferred_element_type=jnp.float32)


_prep_tc = pl.pallas_call(
    _prep_body,
    grid=(_TC_GRID,),
    in_specs=[_plane0_spec, _plane1_spec, _node_spec, _w_spec],
    out_specs=[_node_spec, _node_spec],
    out_shape=[
        jax.ShapeDtypeStruct((NPAD8, 128), jnp.float32),
        jax.ShapeDtypeStruct((NPAD8, 128), jnp.float32),
    ],
)


def _round_body(aa_ref, ab_ref, inv_ref, h_ref, b_ref, wr_ref, wl_ref, h_out, g_out):
    m = (aa_ref[...] + ab_ref[...]) * inv_ref[...]
    hn = m + b_ref[...] + jnp.dot(
        h_ref[...], wr_ref[...], preferred_element_type=jnp.float32
    )
    h_out[...] = hn
    g_out[...] = jnp.dot(hn, wl_ref[...], preferred_element_type=jnp.float32)


_round_tc = pl.pallas_call(
    _round_body,
    grid=(_TC_GRID,),
    in_specs=[_plane0_spec, _plane1_spec, _node_spec, _node_spec, _b_spec, _w_spec, _w_spec],
    out_specs=[_node_spec, _node_spec],
    out_shape=[
        jax.ShapeDtypeStruct((NPAD8, 128), jnp.float32),
        jax.ShapeDtypeStruct((NPAD8, 128), jnp.float32),
    ],
)


def kernel(x, edge_index, W_l, b_l, W_r):
    x_p = jnp.concatenate(
        [x, jnp.zeros((NPAD - N_NODES, D), jnp.float32)]
    )
    x8 = x_p.reshape(NPAD8, 128)

    eye8 = jnp.eye(8, dtype=jnp.float32)
    wl_big = jnp.kron(eye8, W_l.T)
    wr_big = jnp.kron(eye8, W_r.T)
    b_big = jnp.tile(b_l, 8).reshape(1, 128)

    segsum_sc, counts_sc = _sc_kernels()

    cnt2 = counts_sc(edge_index).reshape(2 * NPAD8, 128)
    inv8, g8 = _prep_tc(cnt2, cnt2, x8, wl_big)

    hs = []
    h8 = x8
    for _ in range(3):
        acc2 = segsum_sc(g8.reshape(NPAD, D), edge_index).reshape(2 * NPAD8, 128)
        h8, g8 = _round_tc(
            acc2,
            acc2,
            inv8,
            h8,
            b_big,
            wr_big,
            wl_big,
        )
        hs.append(h8.reshape(NPAD, D)[:N_NODES])

    return jnp.concatenate([x] + hs, axis=-1)
